# initial kernel scaffold (unmeasured)
import jax
import jax.numpy as jnp
from jax import lax
from jax.experimental import pallas as pl
from jax.experimental.pallas import tpu as pltpu


def kernel(
    x,
):
    def body(*refs):
        pass

    out_shape = jax.ShapeDtypeStruct(..., jnp.float32)
    return pl.pallas_call(body, out_shape=out_shape)(...)


# baseline (device time: 17324 ns/iter reference)
import jax
import jax.numpy as jnp
from jax import lax
from jax.experimental import pallas as pl
from jax.experimental.pallas import tpu as pltpu

N_Y = 2


def kernel(x):
    _, m, n_tot = x.shape
    n_half = n_tot // N_Y

    def body(x_ref, out_ref, comm_ref, send_sem, recv_sem):
        my_x = lax.axis_index("x")
        my_y = lax.axis_index("y")
        my_z = lax.axis_index("z")
        peer_y = 1 - my_y
        peer = (my_x, peer_y, my_z)

        barrier_sem = pltpu.get_barrier_semaphore()
        pl.semaphore_signal(
            barrier_sem, inc=1, device_id=peer,
            device_id_type=pl.DeviceIdType.MESH,
        )
        pl.semaphore_wait(barrier_sem, 1)

        rdma = pltpu.make_async_remote_copy(
            src_ref=x_ref.at[0, :, pl.ds(peer_y * n_half, n_half)],
            dst_ref=comm_ref,
            send_sem=send_sem,
            recv_sem=recv_sem,
            device_id=peer,
            device_id_type=pl.DeviceIdType.MESH,
        )
        rdma.start()
        rdma.wait()

        out_ref[:, :] = x_ref[0, :, pl.ds(my_y * n_half, n_half)] + comm_ref[:, :]

    return pl.pallas_call(
        body,
        out_shape=jax.ShapeDtypeStruct((m, n_half), jnp.float32),
        in_specs=[pl.BlockSpec(memory_space=pltpu.VMEM)],
        out_specs=pl.BlockSpec(memory_space=pltpu.VMEM),
        scratch_shapes=[
            pltpu.VMEM((m, n_half), jnp.float32),
            pltpu.SemaphoreType.DMA,
            pltpu.SemaphoreType.DMA,
        ],
        compiler_params=pltpu.CompilerParams(collective_id=0),
    )(x)


# device time: 15069 ns/iter; 1.1496x vs baseline; 1.1496x over previous
import jax
import jax.numpy as jnp
from jax import lax
from jax.experimental import pallas as pl
from jax.experimental.pallas import tpu as pltpu

N_Y = 2
N_X = 2
C = 8


def kernel(x):
    _, m, n_tot = x.shape
    n_half = n_tot // N_Y
    q = n_half // N_X
    r = m // C

    def body(x_ref, out_ref, raw_ref, send1, recv1, send2, recv2):
        my_x = lax.axis_index("x")
        my_y = lax.axis_index("y")
        my_z = lax.axis_index("z")
        y_peer = (my_x, 1 - my_y, my_z)
        x_peer = (1 - my_x, my_y, my_z)

        send_col = (1 - my_y) * n_half + my_x * q
        mine_col = my_y * n_half + my_x * q
        out_col = my_x * q

        barrier_sem = pltpu.get_barrier_semaphore()
        for peer in (y_peer, x_peer):
            pl.semaphore_signal(
                barrier_sem, inc=1, device_id=peer,
                device_id_type=pl.DeviceIdType.MESH,
            )
        pl.semaphore_wait(barrier_sem, 2)

        rdma1 = [
            pltpu.make_async_remote_copy(
                src_ref=x_ref.at[0, pl.ds(c * r, r), pl.ds(send_col, q)],
                dst_ref=raw_ref.at[pl.ds(c * r, r), :],
                send_sem=send1.at[c],
                recv_sem=recv1.at[c],
                device_id=y_peer,
                device_id_type=pl.DeviceIdType.MESH,
            )
            for c in range(C)
        ]
        rdma2 = [
            pltpu.make_async_remote_copy(
                src_ref=out_ref.at[pl.ds(c * r, r), pl.ds(out_col, q)],
                dst_ref=out_ref.at[pl.ds(c * r, r), pl.ds(out_col, q)],
                send_sem=send2.at[c],
                recv_sem=recv2.at[c],
                device_id=x_peer,
                device_id_type=pl.DeviceIdType.MESH,
            )
            for c in range(C)
        ]

        for c in range(C):
            rdma1[c].start()

        for c in range(C):
            rdma1[c].wait_recv()
            rows = pl.ds(c * r, r)
            out_ref[rows, pl.ds(out_col, q)] = (
                x_ref[0, rows, pl.ds(mine_col, q)] + raw_ref[rows, :]
            )
            rdma2[c].start()

        for c in range(C):
            rdma2[c].wait_recv()
            rdma1[c].wait_send()
            rdma2[c].wait_send()

    return pl.pallas_call(
        body,
        out_shape=jax.ShapeDtypeStruct((m, n_half), jnp.float32),
        in_specs=[pl.BlockSpec(memory_space=pltpu.VMEM)],
        out_specs=pl.BlockSpec(memory_space=pltpu.VMEM),
        scratch_shapes=[
            pltpu.VMEM((m, q), jnp.float32),
            pltpu.SemaphoreType.DMA((C,)),
            pltpu.SemaphoreType.DMA((C,)),
            pltpu.SemaphoreType.DMA((C,)),
            pltpu.SemaphoreType.DMA((C,)),
        ],
        compiler_params=pltpu.CompilerParams(collective_id=0),
    )(x)


# device time: 13278 ns/iter; 1.3047x vs baseline; 1.1349x over previous
import jax
import jax.numpy as jnp
from jax import lax
from jax.experimental import pallas as pl
from jax.experimental.pallas import tpu as pltpu

N_Y = 2
N_X = 2
C = 8


def kernel(x):
    _, m, n_tot = x.shape
    n_half = n_tot // N_Y
    q = n_half // N_X
    r = m // C

    def body(x_ref, out_ref, raw_ref, send1, recv1, send2, recv2):
        my_x = lax.axis_index("x")
        my_y = lax.axis_index("y")
        my_z = lax.axis_index("z")
        y_peer = (my_x, 1 - my_y, my_z)
        x_peer = (1 - my_x, my_y, my_z)

        send_col = (1 - my_y) * n_half + my_x * q
        mine_col = my_y * n_half + my_x * q
        out_col = my_x * q

        barrier_sem = pltpu.get_barrier_semaphore()
        for peer in (y_peer, x_peer):
            pl.semaphore_signal(
                barrier_sem, inc=1, device_id=peer,
                device_id_type=pl.DeviceIdType.MESH,
            )
        pl.semaphore_wait(barrier_sem, 2)

        rdma1 = [
            pltpu.make_async_remote_copy(
                src_ref=x_ref.at[0, pl.ds(c * r, r), pl.ds(send_col, q)],
                dst_ref=raw_ref.at[pl.ds(c * r, r), :],
                send_sem=send1.at[c],
                recv_sem=recv1.at[c],
                device_id=y_peer,
                device_id_type=pl.DeviceIdType.MESH,
            )
            for c in range(C)
        ]
        rdma2 = [
            pltpu.make_async_remote_copy(
                src_ref=out_ref.at[pl.ds(c * r, r), pl.ds(out_col, q)],
                dst_ref=out_ref.at[pl.ds(c * r, r), pl.ds(out_col, q)],
                send_sem=send2.at[c],
                recv_sem=recv2.at[c],
                device_id=x_peer,
                device_id_type=pl.DeviceIdType.MESH,
            )
            for c in range(C)
        ]

        for c in range(C):
            rdma1[c].start()
            rdma2[c].start()

        for c in range(C):
            rdma1[c].wait_recv()
            rows = pl.ds(c * r, r)
            out_ref[rows, pl.ds(out_col, q)] = (
                x_ref[0, rows, pl.ds(mine_col, q)] + raw_ref[rows, :]
            )

        for c in range(C):
            rdma2[c].wait_recv()
            rdma1[c].wait_send()
            rdma2[c].wait_send()

    return pl.pallas_call(
        body,
        out_shape=jax.ShapeDtypeStruct((m, n_half), jnp.float32),
        in_specs=[pl.BlockSpec(memory_space=pltpu.VMEM)],
        out_specs=pl.BlockSpec(memory_space=pltpu.VMEM),
        scratch_shapes=[
            pltpu.VMEM((m, q), jnp.float32),
            pltpu.SemaphoreType.DMA((C,)),
            pltpu.SemaphoreType.DMA((C,)),
            pltpu.SemaphoreType.DMA((C,)),
            pltpu.SemaphoreType.DMA((C,)),
        ],
        compiler_params=pltpu.CompilerParams(collective_id=0),
    )(x)
